# baseline (speedup 1.0000x reference)
import functools
import math

import jax
import jax.numpy as jnp
from jax import lax
from jax.experimental import pallas as pl
from jax.experimental.pallas import tpu as pltpu
from jax.experimental.pallas import tpu_sc as plsc

VOCAB = 1000000
D = 64
B = 16384 * 50
NC, NS = 2, 16
NW = NC * NS
PW = B // NW
G = 128
CHUNK = 256
GPC = CHUNK // G
NCHUNK = PW // CHUNK
ROWS_PER_W = PW // G
SCALE = math.sqrt(D)


def _body(table_hbm, idx_hbm, out_hbm, idx_v, rows_v, gsem):
    w = lax.axis_index("s") * NC + lax.axis_index("c")

    pltpu.sync_copy(idx_hbm.at[w], idx_v)

    def chunk_step(g, _):
        base = w * PW + g * CHUNK
        copies = []
        for k in range(GPC):
            copies.append(pltpu.async_copy(
                table_hbm.at[idx_v.at[g * GPC + k]],
                rows_v.at[pl.ds(k * G, G)],
                gsem,
            ))
        for c in copies:
            c.wait()
        def scale_row(r, _):
            for c4 in range(D // 16):
                sl = pl.ds(c4 * 16, 16)
                rows_v[r, sl] = rows_v[r, sl] * SCALE
            return 0
        lax.fori_loop(0, CHUNK, scale_row, 0)
        pltpu.sync_copy(rows_v, out_hbm.at[pl.ds(base, CHUNK)])
        return 0

    lax.fori_loop(0, NCHUNK, chunk_step, 0)


@jax.jit
def kernel(x, table):
    idx = x.reshape(-1).astype(jnp.int32).reshape(NW, ROWS_PER_W, G)
    mesh = plsc.VectorSubcoreMesh(
        core_axis_name="c", subcore_axis_name="s",
        num_cores=NC, num_subcores=NS,
    )
    fn = functools.partial(
        pl.kernel,
        out_type=jax.ShapeDtypeStruct((B, D), jnp.float32),
        mesh=mesh,
        scratch_types=[
            pltpu.VMEM((ROWS_PER_W, G), jnp.int32),
            pltpu.VMEM((CHUNK, D), jnp.float32),
            pltpu.SemaphoreType.DMA,
        ],
        compiler_params=pltpu.CompilerParams(use_tc_tiling_on_sc=False),
    )(_body)
    out = fn(table, idx)
    return out.reshape(16384, 50, D)


# speedup vs baseline: 1.1438x; 1.1438x over previous
import functools
import math

import jax
import jax.numpy as jnp
from jax import lax
from jax.experimental import pallas as pl
from jax.experimental.pallas import tpu as pltpu
from jax.experimental.pallas import tpu_sc as plsc

VOCAB = 1000000
D = 64
B = 16384 * 50
NC, NS = 2, 16
NW = NC * NS
PW = B // NW
G = 128
CHUNK = 256
GPC = CHUNK // G
NCHUNK = PW // CHUNK
GROUPS_PER_W = PW // G
SCALE = math.sqrt(D)


def _body(table_hbm, idx_hbm, out_hbm, idx_v, in_b, out_b,
          gsem0, gsem1, osem0, osem1):
    gsems = (gsem0, gsem1)
    osems = (osem0, osem1)
    w = lax.axis_index("s") * NC + lax.axis_index("c")

    pltpu.sync_copy(idx_hbm.at[w], idx_v)

    def fire_gathers(g, b):
        for k in range(GPC):
            pltpu.make_async_copy(
                table_hbm.at[idx_v.at[g * GPC + k]],
                in_b.at[b].at[pl.ds(k * G, G)],
                gsems[b],
            ).start()

    def wait_gathers(g, b):
        for k in range(GPC):
            pltpu.make_async_copy(
                table_hbm.at[idx_v.at[g * GPC + k]],
                in_b.at[b].at[pl.ds(k * G, G)],
                gsems[b],
            ).wait()

    fire_gathers(0, 0)
    fire_gathers(1, 1)

    def outer(i, _):
        for b in range(2):
            g = 2 * i + b
            base = w * PW + g * CHUNK
            wait_gathers(g, b)

            @pl.when(i >= 1)
            def _():
                pltpu.make_async_copy(
                    out_b.at[b], out_hbm.at[pl.ds(base, CHUNK)], osems[b],
                ).wait()

            @plsc.parallel_loop(0, CHUNK, unroll=4)
            def _(r):
                for c4 in range(D // 16):
                    sl = pl.ds(c4 * 16, 16)
                    out_b[b, r, sl] = in_b[b, r, sl] * SCALE

            @pl.when(i < NCHUNK // 2 - 1)
            def _():
                fire_gathers(g + 2, b)

            pltpu.make_async_copy(
                out_b.at[b], out_hbm.at[pl.ds(base, CHUNK)], osems[b],
            ).start()
        return 0

    lax.fori_loop(0, NCHUNK // 2, outer, 0)

    for b in range(2):
        pltpu.make_async_copy(
            out_b.at[b], out_hbm.at[pl.ds(0, CHUNK)], osems[b],
        ).wait()


@jax.jit
def kernel(x, table):
    idx = x.reshape(-1).astype(jnp.int32).reshape(NW, GROUPS_PER_W, G)
    mesh = plsc.VectorSubcoreMesh(
        core_axis_name="c", subcore_axis_name="s",
        num_cores=NC, num_subcores=NS,
    )
    fn = functools.partial(
        pl.kernel,
        out_type=jax.ShapeDtypeStruct((B, D), jnp.float32),
        mesh=mesh,
        scratch_types=[
            pltpu.VMEM((GROUPS_PER_W, G), jnp.int32),
            pltpu.VMEM((2, CHUNK, D), jnp.float32),
            pltpu.VMEM((2, CHUNK, D), jnp.float32),
            pltpu.SemaphoreType.DMA,
            pltpu.SemaphoreType.DMA,
            pltpu.SemaphoreType.DMA,
            pltpu.SemaphoreType.DMA,
        ],
        compiler_params=pltpu.CompilerParams(use_tc_tiling_on_sc=False),
    )(_body)
    out = fn(table, idx)
    return out.reshape(16384, 50, D)


# speedup vs baseline: 1.1454x; 1.0013x over previous
import functools
import math

import jax
import jax.numpy as jnp
from jax import lax
from jax.experimental import pallas as pl
from jax.experimental.pallas import tpu as pltpu
from jax.experimental.pallas import tpu_sc as plsc

VOCAB = 1000000
D = 64
NSEQ = 16384
SL = 50
NC, NS = 2, 16
NW = NC * NS
SEQ_PW = NSEQ // NW
G = 80
CH_SEQ = 8
CH = CH_SEQ * SL
GPC = CH // G
NCHUNK = SEQ_PW // CH_SEQ
GROUPS_PW = SEQ_PW * SL // G
SCALE = math.sqrt(D)


def _body(table_hbm, idx_hbm, out_hbm, idx_v, in_b, out_b,
          gsem0, gsem1, osem0, osem1):
    gsems = (gsem0, gsem1)
    osems = (osem0, osem1)
    w = lax.axis_index("s") * NC + lax.axis_index("c")

    pltpu.sync_copy(idx_hbm.at[w], idx_v)

    def gather_copies(g, b):
        return [
            pltpu.make_async_copy(
                table_hbm.at[idx_v.at[g * GPC + k]],
                in_b.at[b].at[pl.ds(k * G, G)],
                gsems[b],
            )
            for k in range(GPC)
        ]

    for c in gather_copies(0, 0):
        c.start()
    for c in gather_copies(1, 1):
        c.start()

    def outer(i, _):
        for b in range(2):
            g = 2 * i + b
            seq0 = w * SEQ_PW + g * CH_SEQ
            for c in gather_copies(g, b):
                c.wait()

            @pl.when(i >= 1)
            def _():
                pltpu.make_async_copy(
                    out_b.at[b], out_hbm.at[pl.ds(seq0, CH_SEQ)], osems[b],
                ).wait()

            for si in range(CH_SEQ):
                @plsc.parallel_loop(0, SL, unroll=2)
                def _(r):
                    for c4 in range(D // 16):
                        sl = pl.ds(c4 * 16, 16)
                        out_b[b, si, r, sl] = in_b[b, si * SL + r, sl] * SCALE

            @pl.when(i < NCHUNK // 2 - 1)
            def _():
                for c in gather_copies(g + 2, b):
                    c.start()

            pltpu.make_async_copy(
                out_b.at[b], out_hbm.at[pl.ds(seq0, CH_SEQ)], osems[b],
            ).start()
        return 0

    lax.fori_loop(0, NCHUNK // 2, outer, 0)

    for b in range(2):
        pltpu.make_async_copy(
            out_b.at[b], out_hbm.at[pl.ds(0, CH_SEQ)], osems[b],
        ).wait()


@jax.jit
def kernel(x, table):
    idx = x.reshape(-1).astype(jnp.int32).reshape(NW, GROUPS_PW, G)
    mesh = plsc.VectorSubcoreMesh(
        core_axis_name="c", subcore_axis_name="s",
        num_cores=NC, num_subcores=NS,
    )
    fn = functools.partial(
        pl.kernel,
        out_type=jax.ShapeDtypeStruct((NSEQ, SL, D), jnp.float32),
        mesh=mesh,
        scratch_types=[
            pltpu.VMEM((GROUPS_PW, G), jnp.int32),
            pltpu.VMEM((2, CH, D), jnp.float32),
            pltpu.VMEM((2, CH_SEQ, SL, D), jnp.float32),
            pltpu.SemaphoreType.DMA,
            pltpu.SemaphoreType.DMA,
            pltpu.SemaphoreType.DMA,
            pltpu.SemaphoreType.DMA,
        ],
        compiler_params=pltpu.CompilerParams(use_tc_tiling_on_sc=False),
    )(_body)
    return fn(table, idx)


# speedup vs baseline: 1.1809x; 1.0310x over previous
import functools
import math

import jax
import jax.numpy as jnp
from jax import lax
from jax.experimental import pallas as pl
from jax.experimental.pallas import tpu as pltpu
from jax.experimental.pallas import tpu_sc as plsc

VOCAB = 1000000
D = 64
NSEQ = 16384
SL = 50
NC, NS = 2, 16
NW = NC * NS
SEQ_PW = NSEQ // NW
CH_SEQ = 2
CH = CH_SEQ * SL
NCHUNK = SEQ_PW // CH_SEQ
SCALE = math.sqrt(D)


def _body(table_hbm, idx_hbm, out_hbm, idx_v, idx_half, in_b, out_b,
          gsem0, gsem1, osem0, osem1):
    gsems = (gsem0, gsem1)
    osems = (osem0, osem1)
    w = lax.axis_index("s") * NC + lax.axis_index("c")

    pltpu.sync_copy(idx_hbm.at[w], idx_v)

    zeros16 = jnp.zeros((16,), jnp.int32)

    def prep_and_fire(g, b):
        for k in range(8):
            idx_half[b, pl.ds(k * 16, 16)] = zeros16
        for s in (0, 16, 32, 48, 64, 80, 84):
            sl = pl.ds(s, 16)
            idx_half[b, sl] = idx_v[g, sl] >> 1
        pltpu.make_async_copy(
            table_hbm.at[idx_half.at[b, pl.ds(0, CH)]],
            in_b.at[b],
            gsems[b],
        ).start()

    def wait_gather(b):
        pltpu.make_async_copy(
            table_hbm.at[idx_half.at[b, pl.ds(0, CH)]],
            in_b.at[b],
            gsems[b],
        ).wait()

    prep_and_fire(0, 0)
    prep_and_fire(1, 1)

    def outer(i, _):
        for b in range(2):
            g = 2 * i + b
            seq0 = w * SEQ_PW + g * CH_SEQ
            wait_gather(b)

            @pl.when(i >= 1)
            def _():
                pltpu.make_async_copy(
                    out_b.at[b], out_hbm.at[pl.ds(seq0, CH_SEQ)], osems[b],
                ).wait()

            for si in range(CH_SEQ):
                @plsc.parallel_loop(0, SL, unroll=2)
                def _(r):
                    half = (idx_v[g, pl.ds(si * SL + r, 16)][0] & 1) * D
                    for c4 in range(D // 16):
                        out_b[b, si, r, pl.ds(c4 * 16, 16)] = (
                            in_b[b, si * SL + r, pl.ds(half + c4 * 16, 16)]
                            * SCALE
                        )

            @pl.when(i < NCHUNK // 2 - 1)
            def _():
                prep_and_fire(g + 2, b)

            pltpu.make_async_copy(
                out_b.at[b], out_hbm.at[pl.ds(seq0, CH_SEQ)], osems[b],
            ).start()
        return 0

    lax.fori_loop(0, NCHUNK // 2, outer, 0)

    for b in range(2):
        pltpu.make_async_copy(
            out_b.at[b], out_hbm.at[pl.ds(0, CH_SEQ)], osems[b],
        ).wait()


@jax.jit
def kernel(x, table):
    table2 = table.reshape(VOCAB // 2, 2 * D)
    idx = x.reshape(-1).astype(jnp.int32).reshape(NW, NCHUNK, CH)
    mesh = plsc.VectorSubcoreMesh(
        core_axis_name="c", subcore_axis_name="s",
        num_cores=NC, num_subcores=NS,
    )
    fn = functools.partial(
        pl.kernel,
        out_type=jax.ShapeDtypeStruct((NSEQ, SL, D), jnp.float32),
        mesh=mesh,
        scratch_types=[
            pltpu.VMEM((NCHUNK, CH), jnp.int32),
            pltpu.VMEM((2, 128), jnp.int32),
            pltpu.VMEM((2, CH, 2 * D), jnp.float32),
            pltpu.VMEM((2, CH_SEQ, SL, D), jnp.float32),
            pltpu.SemaphoreType.DMA,
            pltpu.SemaphoreType.DMA,
            pltpu.SemaphoreType.DMA,
            pltpu.SemaphoreType.DMA,
        ],
        compiler_params=pltpu.CompilerParams(use_tc_tiling_on_sc=True),
    )(_body)
    return fn(table2, idx)
